# Initial kernel scaffold; baseline (speedup 1.0000x reference)
import jax
import jax.numpy as jnp
from jax.experimental import pallas as pl


def kernel(node_states, edge_index, edge_type_ids, W_rel, W_root, bias, gamma, beta):
    raise NotImplementedError("write your pallas kernel here")


# speedup vs baseline: 19.3554x; 19.3554x over previous
import functools

import jax
import jax.numpy as jnp
from jax import lax
from jax.experimental import pallas as pl
from jax.experimental.pallas import tpu as pltpu
from jax.experimental.pallas import tpu_sc as plsc

NC = 2
NS = 16
NW = NC * NS
LANES = 16

N = 10000
E = 320000
D = 128
R = 8

N_PAD = 10240
NR_PAD = 81920
CNT_SLICE = NR_PAD // NS
M_SLICE = N_PAD // NS
E_CNT_TILE = E // NS
CCH = 2000
E_MAIN_TILE = E // NW
MCH = 80

BN_H = 1000
BN_EPI = 2000


def _h_table(x, w_rel):
    def body(x_ref, w_ref, h_ref):
        for r in range(R):
            h_ref[r] = jnp.dot(x_ref[...], w_ref[r],
                               preferred_element_type=jnp.float32)

    return pl.pallas_call(
        body,
        grid=(N // BN_H,),
        in_specs=[
            pl.BlockSpec((BN_H, D), lambda i: (i, 0)),
            pl.BlockSpec((R, D, D), lambda i: (0, 0, 0)),
        ],
        out_specs=pl.BlockSpec((R, BN_H, D), lambda i: (0, i, 0)),
        out_shape=jax.ShapeDtypeStruct((R, N, D), jnp.float32),
    )(x, w_rel)


def _sc_aggregate(src, dst, typ, h_table):
    mesh = plsc.VectorSubcoreMesh(core_axis_name="c", subcore_axis_name="s",
                                  num_cores=NC, num_subcores=NS)

    def body(src_hbm, dst_hbm, typ_hbm, h_hbm, mpart_hbm,
             cbuf_v, dstc_v, typc_v, cidx_v, ones_v,
             srcm_v, dstm_v, typm_v, gidx_v, cidxm_v, w_v, rows_v,
             cnt_sh, m_sh, sem):
        c = lax.axis_index("c")
        s = lax.axis_index("s")
        g = c * NS + s

        zeros16 = jnp.zeros((LANES,), jnp.float32)

        def zero_rows(i, carry):
            for j in range(D // LANES):
                rows_v[i, pl.ds(j * LANES, LANES)] = zeros16
            return carry
        lax.fori_loop(0, MCH, zero_rows, 0)

        def zero_cbuf(i, carry):
            cbuf_v[pl.ds(i * LANES, LANES)] = zeros16
            return carry
        lax.fori_loop(0, CNT_SLICE // LANES, zero_cbuf, 0)

        def fill_ones(i, carry):
            ones_v[pl.ds(i * LANES, LANES)] = jnp.full((LANES,), 1.0,
                                                       jnp.float32)
            return carry
        lax.fori_loop(0, CCH // LANES, fill_ones, 0)

        m_row0 = s * M_SLICE
        for k in range(M_SLICE // MCH):
            pltpu.sync_copy(rows_v,
                            m_sh.at[pl.ds(m_row0 + k * MCH, MCH), :])
        pltpu.sync_copy(cbuf_v, cnt_sh.at[pl.ds(s * CNT_SLICE, CNT_SLICE)])

        plsc.subcore_barrier()

        def count_chunk(k, carry):
            base = s * E_CNT_TILE + k * CCH
            pltpu.sync_copy(dst_hbm.at[pl.ds(base, CCH)], dstc_v)
            pltpu.sync_copy(typ_hbm.at[pl.ds(base, CCH)], typc_v)

            def cidx_iter(i, cc):
                dv = dstc_v[pl.ds(i * LANES, LANES)]
                tv = typc_v[pl.ds(i * LANES, LANES)]
                cidx_v[pl.ds(i * LANES, LANES)] = dv * R + tv
                return cc
            lax.fori_loop(0, CCH // LANES, cidx_iter, 0)
            pltpu.sync_copy(ones_v, cnt_sh.at[cidx_v], add=True)
            return carry
        lax.fori_loop(0, E_CNT_TILE // CCH, count_chunk, 0)

        plsc.subcore_barrier()

        pltpu.sync_copy(cnt_sh.at[pl.ds(s * CNT_SLICE, CNT_SLICE)], cbuf_v)

        def recip_iter(i, carry):
            v = cbuf_v[pl.ds(i * LANES, LANES)]
            cbuf_v[pl.ds(i * LANES, LANES)] = 1.0 / jnp.maximum(v, 1.0)
            return carry
        lax.fori_loop(0, CNT_SLICE // LANES, recip_iter, 0)
        pltpu.sync_copy(cbuf_v, cnt_sh.at[pl.ds(s * CNT_SLICE, CNT_SLICE)])

        plsc.subcore_barrier()

        def main_chunk(k, carry):
            base = g * E_MAIN_TILE + k * MCH
            pltpu.sync_copy(src_hbm.at[pl.ds(base, MCH)], srcm_v)
            pltpu.sync_copy(dst_hbm.at[pl.ds(base, MCH)], dstm_v)
            pltpu.sync_copy(typ_hbm.at[pl.ds(base, MCH)], typm_v)

            def idx_iter(i, cc):
                sv = srcm_v[pl.ds(i * LANES, LANES)]
                dv = dstm_v[pl.ds(i * LANES, LANES)]
                tv = typm_v[pl.ds(i * LANES, LANES)]
                gidx_v[pl.ds(i * LANES, LANES)] = tv * N + sv
                cidxm_v[pl.ds(i * LANES, LANES)] = dv * R + tv
                return cc
            lax.fori_loop(0, MCH // LANES, idx_iter, 0)

            pltpu.async_copy(cnt_sh.at[cidxm_v], w_v, sem).wait()
            pltpu.async_copy(h_hbm.at[gidx_v], rows_v, sem).wait()

            def scale_iter(i, cc):
                wv = w_v[pl.ds(i * LANES, LANES)]
                for j in range(LANES):
                    e = i * LANES + j
                    w = wv[j]
                    for k in range(D // LANES):
                        rows_v[e, pl.ds(k * LANES, LANES)] = (
                            rows_v[e, pl.ds(k * LANES, LANES)] * w)
                return cc
            lax.fori_loop(0, MCH // LANES, scale_iter, 0)

            pltpu.sync_copy(rows_v, m_sh.at[dstm_v], add=True)
            return carry
        lax.fori_loop(0, E_MAIN_TILE // MCH, main_chunk, 0)

        plsc.subcore_barrier()

        for k in range(M_SLICE // MCH):
            r0 = m_row0 + k * MCH
            pltpu.sync_copy(m_sh.at[pl.ds(r0, MCH), :], rows_v)
            pltpu.sync_copy(rows_v, mpart_hbm.at[c, pl.ds(r0, MCH), :])

    run = pl.kernel(
        body,
        out_type=jax.ShapeDtypeStruct((NC, N_PAD, D), jnp.float32),
        mesh=mesh,
        scratch_types=[
            pltpu.VMEM((CNT_SLICE,), jnp.float32),
            pltpu.VMEM((CCH,), jnp.int32),
            pltpu.VMEM((CCH,), jnp.int32),
            pltpu.VMEM((CCH,), jnp.int32),
            pltpu.VMEM((CCH,), jnp.float32),
            pltpu.VMEM((MCH,), jnp.int32),
            pltpu.VMEM((MCH,), jnp.int32),
            pltpu.VMEM((MCH,), jnp.int32),
            pltpu.VMEM((MCH,), jnp.int32),
            pltpu.VMEM((MCH,), jnp.int32),
            pltpu.VMEM((MCH,), jnp.float32),
            pltpu.VMEM((MCH, D), jnp.float32),
            pltpu.VMEM_SHARED((NR_PAD,), jnp.float32),
            pltpu.VMEM_SHARED((N_PAD, D), jnp.float32),
            pltpu.SemaphoreType.DMA,
        ],
    )
    return run(src, dst, typ, h_table)


def _epilogue(x, w_root, bias, gamma, beta, m0, m1):
    def body(x_ref, wr_ref, b_ref, g_ref, be_ref, m0_ref, m1_ref, o_ref):
        x = x_ref[...]
        agg = (jnp.dot(x, wr_ref[...], preferred_element_type=jnp.float32)
               + b_ref[...] + m0_ref[...] + m1_ref[...])
        u = 0.5 * agg * (1.0 + lax.erf(agg * (2.0 ** -0.5)))
        y = u + x
        mean = jnp.mean(y, axis=-1, keepdims=True)
        yc = y - mean
        var = jnp.mean(yc * yc, axis=-1, keepdims=True)
        o_ref[...] = (yc * lax.rsqrt(var + 1e-5) * g_ref[...] + be_ref[...])

    return pl.pallas_call(
        body,
        grid=(N // BN_EPI,),
        in_specs=[
            pl.BlockSpec((BN_EPI, D), lambda i: (i, 0)),
            pl.BlockSpec((D, D), lambda i: (0, 0)),
            pl.BlockSpec((1, D), lambda i: (0, 0)),
            pl.BlockSpec((1, D), lambda i: (0, 0)),
            pl.BlockSpec((1, D), lambda i: (0, 0)),
            pl.BlockSpec((BN_EPI, D), lambda i: (i, 0)),
            pl.BlockSpec((BN_EPI, D), lambda i: (i, 0)),
        ],
        out_specs=pl.BlockSpec((BN_EPI, D), lambda i: (i, 0)),
        out_shape=jax.ShapeDtypeStruct((N, D), jnp.float32),
    )(x, w_root, bias.reshape(1, D), gamma.reshape(1, D),
      beta.reshape(1, D), m0, m1)


def kernel(node_states, edge_index, edge_type_ids, W_rel, W_root, bias,
           gamma, beta):
    src = edge_index[0]
    dst = edge_index[1]
    typ = edge_type_ids.astype(jnp.int32)

    h = _h_table(node_states, W_rel).reshape(R * N, D)
    mpart = _sc_aggregate(src, dst, typ, h)
    return _epilogue(node_states, W_root, bias, gamma, beta,
                     mpart[0, :N], mpart[1, :N])


# speedup vs baseline: 37.6627x; 1.9458x over previous
import functools

import jax
import jax.numpy as jnp
from jax import lax
from jax.experimental import pallas as pl
from jax.experimental.pallas import tpu as pltpu
from jax.experimental.pallas import tpu_sc as plsc

NC = 2
NS = 16
NW = NC * NS
LANES = 16

N = 10000
E = 320000
D = 128
R = 8

N_PAD = 10240
NR_PAD = 81920
CNT_SLICE = NR_PAD // NS
M_SLICE = N_PAD // NS
E_CNT_TILE = E // NS
CCH = 2000
E_MAIN_TILE = E // NW
SUP = 2000
MCH = 80
NCHUNK = SUP // MCH
RECIP_CH = 1280

BN_H = 1000
BN_EPI = 2000


def _h_table(x, w_rel):
    def body(x_ref, w_ref, h_ref):
        for r in range(R):
            h_ref[r] = jnp.dot(x_ref[...], w_ref[r],
                               preferred_element_type=jnp.float32)

    return pl.pallas_call(
        body,
        grid=(N // BN_H,),
        in_specs=[
            pl.BlockSpec((BN_H, D), lambda i: (i, 0)),
            pl.BlockSpec((R, D, D), lambda i: (0, 0, 0)),
        ],
        out_specs=pl.BlockSpec((R, BN_H, D), lambda i: (0, i, 0)),
        out_shape=jax.ShapeDtypeStruct((R, N, D), jnp.float32),
    )(x, w_rel)


def _sc_aggregate(src, dst, typ, h_table):
    mesh = plsc.VectorSubcoreMesh(core_axis_name="c", subcore_axis_name="s",
                                  num_cores=NC, num_subcores=NS)

    def body(src_hbm, dst_hbm, typ_hbm, h_hbm, mpart_hbm,
             cbuf_v, srcc_v, dstc_v, typc_v, cidx_v, w_v,
             gidx_v, dsti_v, rows0_v, rows1_v,
             cnt_sh, m_sh, semw, semg0, semg1, sems0, sems1):
        c = lax.axis_index("c")
        s = lax.axis_index("s")
        g = c * NS + s
        rows = (rows0_v, rows1_v)
        semg = (semg0, semg1)
        sems = (sems0, sems1)

        zeros16 = jnp.zeros((LANES,), jnp.float32)

        def zero_rows(i, carry):
            for j in range(D // LANES):
                rows0_v[i, pl.ds(j * LANES, LANES)] = zeros16
            return carry
        lax.fori_loop(0, MCH, zero_rows, 0)

        def zero_cbuf(i, carry):
            cbuf_v[pl.ds(i * LANES, LANES)] = zeros16
            return carry
        lax.fori_loop(0, RECIP_CH // LANES, zero_cbuf, 0)

        def fill_ones(i, carry):
            w_v[pl.ds(i * LANES, LANES)] = jnp.full((LANES,), 1.0,
                                                    jnp.float32)
            return carry
        lax.fori_loop(0, SUP // LANES, fill_ones, 0)

        m_row0 = s * M_SLICE
        for k in range(M_SLICE // MCH):
            pltpu.sync_copy(rows0_v,
                            m_sh.at[pl.ds(m_row0 + k * MCH, MCH), :])
        for k in range(CNT_SLICE // RECIP_CH):
            pltpu.sync_copy(
                cbuf_v,
                cnt_sh.at[pl.ds(s * CNT_SLICE + k * RECIP_CH, RECIP_CH)])

        plsc.subcore_barrier()

        def count_chunk(k, carry):
            base = s * E_CNT_TILE + k * CCH
            pltpu.sync_copy(dst_hbm.at[pl.ds(base, CCH)], dstc_v)
            pltpu.sync_copy(typ_hbm.at[pl.ds(base, CCH)], typc_v)

            def cidx_iter(i, cc):
                dv = dstc_v[pl.ds(i * LANES, LANES)]
                tv = typc_v[pl.ds(i * LANES, LANES)]
                cidx_v[pl.ds(i * LANES, LANES)] = dv * R + tv
                return cc
            lax.fori_loop(0, CCH // LANES, cidx_iter, 0)
            pltpu.sync_copy(w_v, cnt_sh.at[cidx_v], add=True)
            return carry
        lax.fori_loop(0, E_CNT_TILE // CCH, count_chunk, 0)

        plsc.subcore_barrier()

        for k in range(CNT_SLICE // RECIP_CH):
            off = s * CNT_SLICE + k * RECIP_CH
            pltpu.sync_copy(cnt_sh.at[pl.ds(off, RECIP_CH)], cbuf_v)

            def recip_iter(i, carry):
                v = cbuf_v[pl.ds(i * LANES, LANES)]
                cbuf_v[pl.ds(i * LANES, LANES)] = 1.0 / jnp.maximum(v, 1.0)
                return carry
            lax.fori_loop(0, RECIP_CH // LANES, recip_iter, 0)
            pltpu.sync_copy(cbuf_v, cnt_sh.at[pl.ds(off, RECIP_CH)])

        plsc.subcore_barrier()

        def scale_chunk(b, k):
            def scale_iter(i, cc):
                wv = w_v[pl.ds(k * MCH + i * LANES, LANES)]
                for j in range(LANES):
                    e = i * LANES + j
                    w = wv[j]
                    for q in range(D // LANES):
                        rows[b][e, pl.ds(q * LANES, LANES)] = (
                            rows[b][e, pl.ds(q * LANES, LANES)] * w)
                return cc
            lax.fori_loop(0, MCH // LANES, scale_iter, 0)

        def superblock(sb, carry):
            base = g * E_MAIN_TILE + sb * SUP
            pltpu.sync_copy(src_hbm.at[pl.ds(base, SUP)], srcc_v)
            pltpu.sync_copy(dst_hbm.at[pl.ds(base, SUP)], dstc_v)
            pltpu.sync_copy(typ_hbm.at[pl.ds(base, SUP)], typc_v)

            def idx_iter(k, cc):
                for j in range(MCH // LANES):
                    off = k * MCH + j * LANES
                    sv = srcc_v[pl.ds(off, LANES)]
                    dv = dstc_v[pl.ds(off, LANES)]
                    tv = typc_v[pl.ds(off, LANES)]
                    gidx_v[k, pl.ds(j * LANES, LANES)] = tv * N + sv
                    dsti_v[k, pl.ds(j * LANES, LANES)] = dv
                    cidx_v[pl.ds(off, LANES)] = dv * R + tv
                return cc
            lax.fori_loop(0, NCHUNK, idx_iter, 0)

            pltpu.async_copy(cnt_sh.at[cidx_v], w_v, semw).wait()

            gather_d = [None, None]
            scatter_d = [None, None]
            gather_d[0] = pltpu.async_copy(
                h_hbm.at[gidx_v.at[0]], rows[0], semg[0])
            for k in range(NCHUNK):
                b = k % 2
                nb = (k + 1) % 2
                if k + 1 < NCHUNK:
                    if scatter_d[nb] is not None:
                        scatter_d[nb].wait()
                        scatter_d[nb] = None
                    gather_d[nb] = pltpu.async_copy(
                        h_hbm.at[gidx_v.at[k + 1]], rows[nb], semg[nb])
                gather_d[b].wait()
                scale_chunk(b, k)
                scatter_d[b] = pltpu.async_copy(
                    rows[b], m_sh.at[dsti_v.at[k]], sems[b], add=True)
            for d in scatter_d:
                if d is not None:
                    d.wait()
            return carry
        lax.fori_loop(0, E_MAIN_TILE // SUP, superblock, 0)

        plsc.subcore_barrier()

        for k in range(M_SLICE // MCH):
            r0 = m_row0 + k * MCH
            pltpu.sync_copy(m_sh.at[pl.ds(r0, MCH), :], rows0_v)
            pltpu.sync_copy(rows0_v, mpart_hbm.at[c, pl.ds(r0, MCH), :])

    run = pl.kernel(
        body,
        out_type=jax.ShapeDtypeStruct((NC, N_PAD, D), jnp.float32),
        mesh=mesh,
        scratch_types=[
            pltpu.VMEM((RECIP_CH,), jnp.float32),
            pltpu.VMEM((SUP,), jnp.int32),
            pltpu.VMEM((SUP,), jnp.int32),
            pltpu.VMEM((SUP,), jnp.int32),
            pltpu.VMEM((SUP,), jnp.int32),
            pltpu.VMEM((SUP,), jnp.float32),
            pltpu.VMEM((NCHUNK, MCH), jnp.int32),
            pltpu.VMEM((NCHUNK, MCH), jnp.int32),
            pltpu.VMEM((MCH, D), jnp.float32),
            pltpu.VMEM((MCH, D), jnp.float32),
            pltpu.VMEM_SHARED((NR_PAD,), jnp.float32),
            pltpu.VMEM_SHARED((N_PAD, D), jnp.float32),
            pltpu.SemaphoreType.DMA,
            pltpu.SemaphoreType.DMA,
            pltpu.SemaphoreType.DMA,
            pltpu.SemaphoreType.DMA,
            pltpu.SemaphoreType.DMA,
        ],
    )
    return run(src, dst, typ, h_table)


def _epilogue(x, w_root, bias, gamma, beta, m0, m1):
    def body(x_ref, wr_ref, b_ref, g_ref, be_ref, m0_ref, m1_ref, o_ref):
        x = x_ref[...]
        agg = (jnp.dot(x, wr_ref[...], preferred_element_type=jnp.float32)
               + b_ref[...] + m0_ref[...] + m1_ref[...])
        u = 0.5 * agg * (1.0 + lax.erf(agg * (2.0 ** -0.5)))
        y = u + x
        mean = jnp.mean(y, axis=-1, keepdims=True)
        yc = y - mean
        var = jnp.mean(yc * yc, axis=-1, keepdims=True)
        o_ref[...] = (yc * lax.rsqrt(var + 1e-5) * g_ref[...] + be_ref[...])

    return pl.pallas_call(
        body,
        grid=(N // BN_EPI,),
        in_specs=[
            pl.BlockSpec((BN_EPI, D), lambda i: (i, 0)),
            pl.BlockSpec((D, D), lambda i: (0, 0)),
            pl.BlockSpec((1, D), lambda i: (0, 0)),
            pl.BlockSpec((1, D), lambda i: (0, 0)),
            pl.BlockSpec((1, D), lambda i: (0, 0)),
            pl.BlockSpec((BN_EPI, D), lambda i: (i, 0)),
            pl.BlockSpec((BN_EPI, D), lambda i: (i, 0)),
        ],
        out_specs=pl.BlockSpec((BN_EPI, D), lambda i: (i, 0)),
        out_shape=jax.ShapeDtypeStruct((N, D), jnp.float32),
    )(x, w_root, bias.reshape(1, D), gamma.reshape(1, D),
      beta.reshape(1, D), m0, m1)


def kernel(node_states, edge_index, edge_type_ids, W_rel, W_root, bias,
           gamma, beta):
    src = edge_index[0]
    dst = edge_index[1]
    typ = edge_type_ids.astype(jnp.int32)

    h = _h_table(node_states, W_rel).reshape(R * N, D)
    mpart = _sc_aggregate(src, dst, typ, h)
    return _epilogue(node_states, W_root, bias, gamma, beta,
                     mpart[0, :N], mpart[1, :N])


# speedup vs baseline: 39.2213x; 1.0414x over previous
import functools

import jax
import jax.numpy as jnp
from jax import lax
from jax.experimental import pallas as pl
from jax.experimental.pallas import tpu as pltpu
from jax.experimental.pallas import tpu_sc as plsc

NC = 2
NS = 16
NW = NC * NS
LANES = 16

N = 10000
E = 320000
D = 128
R = 8

N_PAD = 10240
NR_PAD = 81920
CNT_SLICE = NR_PAD // NS
M_SLICE = N_PAD // NS
E_CNT_TILE = E // NS
CCH = 2000
E_MAIN_TILE = E // NW
SUP = 2000
MCH = 80
NCHUNK = SUP // MCH
RECIP_CH = 1280

BN_H = 1000
BN_EPI = 2000


def _h_table(x, w_rel):
    def body(x_ref, w_ref, h_ref):
        for r in range(R):
            h_ref[r] = jnp.dot(x_ref[...], w_ref[r],
                               preferred_element_type=jnp.float32)

    return pl.pallas_call(
        body,
        grid=(N // BN_H,),
        in_specs=[
            pl.BlockSpec((BN_H, D), lambda i: (i, 0)),
            pl.BlockSpec((R, D, D), lambda i: (0, 0, 0)),
        ],
        out_specs=pl.BlockSpec((R, BN_H, D), lambda i: (0, i, 0)),
        out_shape=jax.ShapeDtypeStruct((R, N, D), jnp.float32),
    )(x, w_rel)


def _sc_aggregate(src, dst, typ, h_table):
    mesh = plsc.VectorSubcoreMesh(core_axis_name="c", subcore_axis_name="s",
                                  num_cores=NC, num_subcores=NS)

    def body(src_hbm, dst_hbm, typ_hbm, h_hbm, mpart_hbm,
             cbuf_v, srcc_v, dstc_v, typc_v, cidx_v, w_v,
             gidx_v, dsti_v, rows0_v, rows1_v,
             cnt_sh, m_sh, semw, semg0, semg1, sems0, sems1, seme):
        c = lax.axis_index("c")
        s = lax.axis_index("s")
        g = c * NS + s
        rows = (rows0_v, rows1_v)
        semg = (semg0, semg1)
        sems = (sems0, sems1)

        zeros16 = jnp.zeros((LANES,), jnp.float32)

        def zero_rows(i, carry):
            for j in range(D // LANES):
                rows0_v[i, pl.ds(j * LANES, LANES)] = zeros16
            return carry
        lax.fori_loop(0, MCH, zero_rows, 0)

        def zero_cbuf(i, carry):
            cbuf_v[pl.ds(i * LANES, LANES)] = zeros16
            return carry
        lax.fori_loop(0, RECIP_CH // LANES, zero_cbuf, 0)

        def fill_ones(i, carry):
            w_v[pl.ds(i * LANES, LANES)] = jnp.full((LANES,), 1.0,
                                                    jnp.float32)
            return carry
        lax.fori_loop(0, SUP // LANES, fill_ones, 0)

        m_row0 = s * M_SLICE
        for k in range(M_SLICE // MCH):
            pltpu.sync_copy(rows0_v,
                            m_sh.at[pl.ds(m_row0 + k * MCH, MCH), :])
        for k in range(CNT_SLICE // RECIP_CH):
            pltpu.sync_copy(
                cbuf_v,
                cnt_sh.at[pl.ds(s * CNT_SLICE + k * RECIP_CH, RECIP_CH)])

        plsc.subcore_barrier()

        def count_chunk(k, carry):
            base = s * E_CNT_TILE + k * CCH
            pltpu.sync_copy(dst_hbm.at[pl.ds(base, CCH)], dstc_v)
            pltpu.sync_copy(typ_hbm.at[pl.ds(base, CCH)], typc_v)

            def cidx_iter(i, cc):
                dv = dstc_v[pl.ds(i * LANES, LANES)]
                tv = typc_v[pl.ds(i * LANES, LANES)]
                cidx_v[pl.ds(i * LANES, LANES)] = dv * R + tv
                return cc
            lax.fori_loop(0, CCH // LANES, cidx_iter, 0)
            pltpu.sync_copy(w_v, cnt_sh.at[cidx_v], add=True)
            return carry
        lax.fori_loop(0, E_CNT_TILE // CCH, count_chunk, 0)

        plsc.subcore_barrier()

        for k in range(CNT_SLICE // RECIP_CH):
            off = s * CNT_SLICE + k * RECIP_CH
            pltpu.sync_copy(cnt_sh.at[pl.ds(off, RECIP_CH)], cbuf_v)

            def recip_iter(i, carry):
                v = cbuf_v[pl.ds(i * LANES, LANES)]
                cbuf_v[pl.ds(i * LANES, LANES)] = 1.0 / jnp.maximum(v, 1.0)
                return carry
            lax.fori_loop(0, RECIP_CH // LANES, recip_iter, 0)
            pltpu.sync_copy(cbuf_v, cnt_sh.at[pl.ds(off, RECIP_CH)])

        plsc.subcore_barrier()

        def scale_chunk(b, k):
            def scale_iter(i, cc):
                wv = w_v[pl.ds(k * MCH + i * LANES, LANES)]
                for j in range(LANES):
                    e = i * LANES + j
                    w = wv[j]
                    for q in range(D // LANES):
                        rows[b][e, pl.ds(q * LANES, LANES)] = (
                            rows[b][e, pl.ds(q * LANES, LANES)] * w)
                return cc
            lax.fori_loop(0, MCH // LANES, scale_iter, 0)

        n_sup = E_MAIN_TILE // SUP
        base0 = g * E_MAIN_TILE
        pltpu.async_copy(src_hbm.at[pl.ds(base0, SUP)], srcc_v, seme)
        pltpu.async_copy(dst_hbm.at[pl.ds(base0, SUP)], dstc_v, seme)
        pltpu.async_copy(typ_hbm.at[pl.ds(base0, SUP)], typc_v, seme)

        def superblock(sb, carry):
            pltpu.make_async_copy(src_hbm.at[pl.ds(0, SUP)], srcc_v,
                                  seme).wait()
            pltpu.make_async_copy(dst_hbm.at[pl.ds(0, SUP)], dstc_v,
                                  seme).wait()
            pltpu.make_async_copy(typ_hbm.at[pl.ds(0, SUP)], typc_v,
                                  seme).wait()

            def idx_iter(k, cc):
                for j in range(MCH // LANES):
                    off = k * MCH + j * LANES
                    sv = srcc_v[pl.ds(off, LANES)]
                    dv = dstc_v[pl.ds(off, LANES)]
                    tv = typc_v[pl.ds(off, LANES)]
                    gidx_v[k, pl.ds(j * LANES, LANES)] = tv * N + sv
                    dsti_v[k, pl.ds(j * LANES, LANES)] = dv
                    cidx_v[pl.ds(off, LANES)] = dv * R + tv
                return cc
            lax.fori_loop(0, NCHUNK, idx_iter, 0)

            @pl.when(sb + 1 < n_sup)
            def _prefetch():
                nbase = base0 + (sb + 1) * SUP
                pltpu.async_copy(src_hbm.at[pl.ds(nbase, SUP)], srcc_v, seme)
                pltpu.async_copy(dst_hbm.at[pl.ds(nbase, SUP)], dstc_v, seme)
                pltpu.async_copy(typ_hbm.at[pl.ds(nbase, SUP)], typc_v, seme)

            w_d = pltpu.async_copy(cnt_sh.at[cidx_v], w_v, semw)

            gather_d = [None, None]
            scatter_d = [None, None]
            gather_d[0] = pltpu.async_copy(
                h_hbm.at[gidx_v.at[0]], rows[0], semg[0])
            w_d.wait()
            for k in range(NCHUNK):
                b = k % 2
                nb = (k + 1) % 2
                if k + 1 < NCHUNK:
                    if scatter_d[nb] is not None:
                        scatter_d[nb].wait()
                        scatter_d[nb] = None
                    gather_d[nb] = pltpu.async_copy(
                        h_hbm.at[gidx_v.at[k + 1]], rows[nb], semg[nb])
                gather_d[b].wait()
                scale_chunk(b, k)
                scatter_d[b] = pltpu.async_copy(
                    rows[b], m_sh.at[dsti_v.at[k]], sems[b], add=True)
            for d in scatter_d:
                if d is not None:
                    d.wait()
            return carry
        lax.fori_loop(0, E_MAIN_TILE // SUP, superblock, 0)

        plsc.subcore_barrier()

        pltpu.sync_copy(m_sh.at[pl.ds(m_row0, M_SLICE), :],
                        mpart_hbm.at[c, pl.ds(m_row0, M_SLICE), :])

    run = pl.kernel(
        body,
        out_type=jax.ShapeDtypeStruct((NC, N_PAD, D), jnp.float32),
        mesh=mesh,
        scratch_types=[
            pltpu.VMEM((RECIP_CH,), jnp.float32),
            pltpu.VMEM((SUP,), jnp.int32),
            pltpu.VMEM((SUP,), jnp.int32),
            pltpu.VMEM((SUP,), jnp.int32),
            pltpu.VMEM((SUP,), jnp.int32),
            pltpu.VMEM((SUP,), jnp.float32),
            pltpu.VMEM((NCHUNK, MCH), jnp.int32),
            pltpu.VMEM((NCHUNK, MCH), jnp.int32),
            pltpu.VMEM((MCH, D), jnp.float32),
            pltpu.VMEM((MCH, D), jnp.float32),
            pltpu.VMEM_SHARED((NR_PAD,), jnp.float32),
            pltpu.VMEM_SHARED((N_PAD, D), jnp.float32),
            pltpu.SemaphoreType.DMA,
            pltpu.SemaphoreType.DMA,
            pltpu.SemaphoreType.DMA,
            pltpu.SemaphoreType.DMA,
            pltpu.SemaphoreType.DMA,
            pltpu.SemaphoreType.DMA,
        ],
    )
    return run(src, dst, typ, h_table)


def _epilogue(x, w_root, bias, gamma, beta, m0, m1):
    def body(x_ref, wr_ref, b_ref, g_ref, be_ref, m0_ref, m1_ref, o_ref):
        x = x_ref[...]
        agg = (jnp.dot(x, wr_ref[...], preferred_element_type=jnp.float32)
               + b_ref[...] + m0_ref[...] + m1_ref[...])
        u = 0.5 * agg * (1.0 + lax.erf(agg * (2.0 ** -0.5)))
        y = u + x
        mean = jnp.mean(y, axis=-1, keepdims=True)
        yc = y - mean
        var = jnp.mean(yc * yc, axis=-1, keepdims=True)
        o_ref[...] = (yc * lax.rsqrt(var + 1e-5) * g_ref[...] + be_ref[...])

    return pl.pallas_call(
        body,
        grid=(N // BN_EPI,),
        in_specs=[
            pl.BlockSpec((BN_EPI, D), lambda i: (i, 0)),
            pl.BlockSpec((D, D), lambda i: (0, 0)),
            pl.BlockSpec((1, D), lambda i: (0, 0)),
            pl.BlockSpec((1, D), lambda i: (0, 0)),
            pl.BlockSpec((1, D), lambda i: (0, 0)),
            pl.BlockSpec((BN_EPI, D), lambda i: (i, 0)),
            pl.BlockSpec((BN_EPI, D), lambda i: (i, 0)),
        ],
        out_specs=pl.BlockSpec((BN_EPI, D), lambda i: (i, 0)),
        out_shape=jax.ShapeDtypeStruct((N, D), jnp.float32),
    )(x, w_root, bias.reshape(1, D), gamma.reshape(1, D),
      beta.reshape(1, D), m0, m1)


def kernel(node_states, edge_index, edge_type_ids, W_rel, W_root, bias,
           gamma, beta):
    src = edge_index[0]
    dst = edge_index[1]
    typ = edge_type_ids.astype(jnp.int32)

    h = _h_table(node_states, W_rel).reshape(R * N, D)
    mpart = _sc_aggregate(src, dst, typ, h)
    return _epilogue(node_states, W_root, bias, gamma, beta,
                     mpart[0, :N], mpart[1, :N])


# speedup vs baseline: 41.4619x; 1.0571x over previous
import functools

import jax
import jax.numpy as jnp
from jax import lax
from jax.experimental import pallas as pl
from jax.experimental.pallas import tpu as pltpu
from jax.experimental.pallas import tpu_sc as plsc

NC = 2
NS = 16
NW = NC * NS
LANES = 16

N = 10000
E = 320000
D = 128
R = 8

N_PAD = 10240
NR_PAD = 81920
CNT_SLICE = NR_PAD // NS
M_SLICE = N_PAD // NS
E_CNT_TILE = E // NS
CCH = 2000
E_MAIN_TILE = E // NW
SUP = 2000
MCH = 80
NCHUNK = SUP // MCH
NBUF = 3
RECIP_CH = 1024

BN_H = 1000
BN_EPI = 2000


def _h_table(x, w_rel):
    def body(x_ref, w_ref, h_ref):
        for r in range(R):
            h_ref[:, r, :] = jnp.dot(x_ref[...], w_ref[r],
                                     preferred_element_type=jnp.float32)

    return pl.pallas_call(
        body,
        grid=(N // BN_H,),
        in_specs=[
            pl.BlockSpec((BN_H, D), lambda i: (i, 0)),
            pl.BlockSpec((R, D, D), lambda i: (0, 0, 0)),
        ],
        out_specs=pl.BlockSpec((BN_H, R, D), lambda i: (i, 0, 0)),
        out_shape=jax.ShapeDtypeStruct((N, R, D), jnp.float32),
    )(x, w_rel)


def _sc_counts(dst, typ):
    mesh = plsc.VectorSubcoreMesh(core_axis_name="c", subcore_axis_name="s",
                                  num_cores=NC, num_subcores=NS)

    def body(dst_hbm, typ_hbm, recip_hbm,
             cbuf_v, dstc_v, typc_v, cidx_v, ones_v, cnt_sh):
        c = lax.axis_index("c")
        s = lax.axis_index("s")

        zeros16 = jnp.zeros((LANES,), jnp.float32)

        def zero_cbuf(i, carry):
            cbuf_v[pl.ds(i * LANES, LANES)] = zeros16
            return carry
        lax.fori_loop(0, RECIP_CH // LANES, zero_cbuf, 0)

        def fill_ones(i, carry):
            ones_v[pl.ds(i * LANES, LANES)] = jnp.full((LANES,), 1.0,
                                                       jnp.float32)
            return carry
        lax.fori_loop(0, CCH // LANES, fill_ones, 0)

        for k in range(CNT_SLICE // RECIP_CH):
            pltpu.sync_copy(
                cbuf_v,
                cnt_sh.at[pl.ds(s * CNT_SLICE + k * RECIP_CH, RECIP_CH)])

        plsc.subcore_barrier()

        def count_chunk(k, carry):
            base = s * E_CNT_TILE + k * CCH
            pltpu.sync_copy(dst_hbm.at[pl.ds(base, CCH)], dstc_v)
            pltpu.sync_copy(typ_hbm.at[pl.ds(base, CCH)], typc_v)

            def cidx_iter(i, cc):
                dv = dstc_v[pl.ds(i * LANES, LANES)]
                tv = typc_v[pl.ds(i * LANES, LANES)]
                cidx_v[pl.ds(i * LANES, LANES)] = dv * R + tv
                return cc
            lax.fori_loop(0, CCH // LANES, cidx_iter, 0)
            pltpu.sync_copy(ones_v, cnt_sh.at[cidx_v], add=True)
            return carry
        lax.fori_loop(0, E_CNT_TILE // CCH, count_chunk, 0)

        plsc.subcore_barrier()

        for k in range(CNT_SLICE // RECIP_CH):
            off = s * CNT_SLICE + k * RECIP_CH
            pltpu.sync_copy(cnt_sh.at[pl.ds(off, RECIP_CH)], cbuf_v)

            def recip_iter(i, carry):
                v = cbuf_v[pl.ds(i * LANES, LANES)]
                cbuf_v[pl.ds(i * LANES, LANES)] = 1.0 / jnp.maximum(v, 1.0)
                return carry
            lax.fori_loop(0, RECIP_CH // LANES, recip_iter, 0)
            pltpu.sync_copy(cbuf_v, cnt_sh.at[pl.ds(off, RECIP_CH)])

        plsc.subcore_barrier()
        pltpu.sync_copy(
            cnt_sh.at[pl.ds(s * CNT_SLICE, CNT_SLICE)],
            recip_hbm.at[pl.ds(c * NR_PAD + s * CNT_SLICE, CNT_SLICE)])

    run = pl.kernel(
        body,
        out_type=jax.ShapeDtypeStruct((NC * NR_PAD,), jnp.float32),
        mesh=mesh,
        scratch_types=[
            pltpu.VMEM((RECIP_CH,), jnp.float32),
            pltpu.VMEM((CCH,), jnp.int32),
            pltpu.VMEM((CCH,), jnp.int32),
            pltpu.VMEM((CCH,), jnp.int32),
            pltpu.VMEM((CCH,), jnp.float32),
            pltpu.VMEM_SHARED((NR_PAD,), jnp.float32),
        ],
    )
    return run(dst, typ)


def _sc_aggregate(src, dst, typ, h_table, recip):
    mesh = plsc.VectorSubcoreMesh(core_axis_name="c", subcore_axis_name="s",
                                  num_cores=NC, num_subcores=NS)

    def body(src_hbm, dst_hbm, typ_hbm, h_hbm, recip_hbm, mpart_hbm,
             srcc_v, dstc_v, typc_v, cidx_v, w_v,
             gidx_v, dsti_v, rows0_v, rows1_v, rows2_v,
             m_sh, semw, semg0, semg1, semg2, sems0, sems1, sems2, seme):
        c = lax.axis_index("c")
        s = lax.axis_index("s")
        g = c * NS + s
        rows = (rows0_v, rows1_v, rows2_v)
        semg = (semg0, semg1, semg2)
        sems = (sems0, sems1, sems2)

        zeros16 = jnp.zeros((LANES,), jnp.float32)

        def zero_rows(i, carry):
            for j in range(D // LANES):
                rows0_v[i, pl.ds(j * LANES, LANES)] = zeros16
            return carry
        lax.fori_loop(0, MCH, zero_rows, 0)

        m_row0 = s * M_SLICE
        for k in range(M_SLICE // MCH):
            pltpu.sync_copy(rows0_v,
                            m_sh.at[pl.ds(m_row0 + k * MCH, MCH), :])

        plsc.subcore_barrier()

        def scale_chunk(b, k):
            def scale_iter(i, cc):
                wv = w_v[pl.ds(k * MCH + i * LANES, LANES)]
                for j in range(LANES):
                    e = i * LANES + j
                    w = wv[j]
                    for q in range(D // LANES):
                        rows[b][e, pl.ds(q * LANES, LANES)] = (
                            rows[b][e, pl.ds(q * LANES, LANES)] * w)
                return cc
            lax.fori_loop(0, MCH // LANES, scale_iter, 0)

        n_sup = E_MAIN_TILE // SUP
        base0 = g * E_MAIN_TILE
        pltpu.async_copy(src_hbm.at[pl.ds(base0, SUP)], srcc_v, seme)
        pltpu.async_copy(dst_hbm.at[pl.ds(base0, SUP)], dstc_v, seme)
        pltpu.async_copy(typ_hbm.at[pl.ds(base0, SUP)], typc_v, seme)

        def superblock(sb, carry):
            pltpu.make_async_copy(src_hbm.at[pl.ds(0, SUP)], srcc_v,
                                  seme).wait()
            pltpu.make_async_copy(dst_hbm.at[pl.ds(0, SUP)], dstc_v,
                                  seme).wait()
            pltpu.make_async_copy(typ_hbm.at[pl.ds(0, SUP)], typc_v,
                                  seme).wait()

            coff = c * NR_PAD

            def idx_iter(k, cc):
                for j in range(MCH // LANES):
                    off = k * MCH + j * LANES
                    sv = srcc_v[pl.ds(off, LANES)]
                    dv = dstc_v[pl.ds(off, LANES)]
                    tv = typc_v[pl.ds(off, LANES)]
                    gidx_v[k, pl.ds(j * LANES, LANES)] = sv * R + tv
                    dsti_v[k, pl.ds(j * LANES, LANES)] = dv
                    cidx_v[pl.ds(off, LANES)] = dv * R + tv + coff
                return cc
            lax.fori_loop(0, NCHUNK, idx_iter, 0)

            @pl.when(sb + 1 < n_sup)
            def _prefetch():
                nbase = base0 + (sb + 1) * SUP
                pltpu.async_copy(src_hbm.at[pl.ds(nbase, SUP)], srcc_v, seme)
                pltpu.async_copy(dst_hbm.at[pl.ds(nbase, SUP)], dstc_v, seme)
                pltpu.async_copy(typ_hbm.at[pl.ds(nbase, SUP)], typc_v, seme)

            w_d = pltpu.async_copy(recip_hbm.at[cidx_v], w_v, semw)

            gather_d = [None] * NBUF
            scatter_d = [None] * NBUF
            gather_d[0] = pltpu.async_copy(
                h_hbm.at[gidx_v.at[0]], rows[0], semg[0])
            gather_d[1] = pltpu.async_copy(
                h_hbm.at[gidx_v.at[1]], rows[1], semg[1])
            w_d.wait()
            for k in range(NCHUNK):
                b = k % NBUF
                nb = (k + 2) % NBUF
                if k + 2 < NCHUNK:
                    if scatter_d[nb] is not None:
                        scatter_d[nb].wait()
                        scatter_d[nb] = None
                    gather_d[nb] = pltpu.async_copy(
                        h_hbm.at[gidx_v.at[k + 2]], rows[nb], semg[nb])
                gather_d[b].wait()
                scale_chunk(b, k)
                scatter_d[b] = pltpu.async_copy(
                    rows[b], m_sh.at[dsti_v.at[k]], sems[b], add=True)
            for d in scatter_d:
                if d is not None:
                    d.wait()
            return carry
        lax.fori_loop(0, n_sup, superblock, 0)

        plsc.subcore_barrier()

        pltpu.sync_copy(m_sh.at[pl.ds(m_row0, M_SLICE), :],
                        mpart_hbm.at[c, pl.ds(m_row0, M_SLICE), :])

    run = pl.kernel(
        body,
        out_type=jax.ShapeDtypeStruct((NC, N_PAD, D), jnp.float32),
        mesh=mesh,
        scratch_types=[
            pltpu.VMEM((SUP,), jnp.int32),
            pltpu.VMEM((SUP,), jnp.int32),
            pltpu.VMEM((SUP,), jnp.int32),
            pltpu.VMEM((SUP,), jnp.int32),
            pltpu.VMEM((SUP,), jnp.float32),
            pltpu.VMEM((NCHUNK, MCH), jnp.int32),
            pltpu.VMEM((NCHUNK, MCH), jnp.int32),
            pltpu.VMEM((MCH, D), jnp.float32),
            pltpu.VMEM((MCH, D), jnp.float32),
            pltpu.VMEM((MCH, D), jnp.float32),
            pltpu.VMEM_SHARED((N_PAD, D), jnp.float32),
            pltpu.SemaphoreType.DMA,
            pltpu.SemaphoreType.DMA,
            pltpu.SemaphoreType.DMA,
            pltpu.SemaphoreType.DMA,
            pltpu.SemaphoreType.DMA,
            pltpu.SemaphoreType.DMA,
            pltpu.SemaphoreType.DMA,
            pltpu.SemaphoreType.DMA,
        ],
    )
    return run(src, dst, typ, h_table, recip)


def _epilogue(x, w_root, bias, gamma, beta, m0, m1):
    def body(x_ref, wr_ref, b_ref, g_ref, be_ref, m0_ref, m1_ref, o_ref):
        x = x_ref[...]
        agg = (jnp.dot(x, wr_ref[...], preferred_element_type=jnp.float32)
               + b_ref[...] + m0_ref[...] + m1_ref[...])
        u = 0.5 * agg * (1.0 + lax.erf(agg * (2.0 ** -0.5)))
        y = u + x
        mean = jnp.mean(y, axis=-1, keepdims=True)
        yc = y - mean
        var = jnp.mean(yc * yc, axis=-1, keepdims=True)
        o_ref[...] = (yc * lax.rsqrt(var + 1e-5) * g_ref[...] + be_ref[...])

    return pl.pallas_call(
        body,
        grid=(N // BN_EPI,),
        in_specs=[
            pl.BlockSpec((BN_EPI, D), lambda i: (i, 0)),
            pl.BlockSpec((D, D), lambda i: (0, 0)),
            pl.BlockSpec((1, D), lambda i: (0, 0)),
            pl.BlockSpec((1, D), lambda i: (0, 0)),
            pl.BlockSpec((1, D), lambda i: (0, 0)),
            pl.BlockSpec((BN_EPI, D), lambda i: (i, 0)),
            pl.BlockSpec((BN_EPI, D), lambda i: (i, 0)),
        ],
        out_specs=pl.BlockSpec((BN_EPI, D), lambda i: (i, 0)),
        out_shape=jax.ShapeDtypeStruct((N, D), jnp.float32),
    )(x, w_root, bias.reshape(1, D), gamma.reshape(1, D),
      beta.reshape(1, D), m0, m1)


def kernel(node_states, edge_index, edge_type_ids, W_rel, W_root, bias,
           gamma, beta):
    src = edge_index[0]
    dst = edge_index[1]
    typ = edge_type_ids.astype(jnp.int32)

    h = _h_table(node_states, W_rel).reshape(N * R, D)
    recip = _sc_counts(dst, typ)
    mpart = _sc_aggregate(src, dst, typ, h, recip)
    return _epilogue(node_states, W_root, bias, gamma, beta,
                     mpart[0, :N], mpart[1, :N])


# speedup vs baseline: 43.8145x; 1.0567x over previous
import functools

import jax
import jax.numpy as jnp
from jax import lax
from jax.experimental import pallas as pl
from jax.experimental.pallas import tpu as pltpu
from jax.experimental.pallas import tpu_sc as plsc

NC = 2
NS = 16
NW = NC * NS
LANES = 16

N = 10000
E = 320000
D = 128
R = 8

N_PAD = 10240
NR_PAD = 81920
CNT_SLICE = NR_PAD // NS
M_SLICE = N_PAD // NS
E_CNT_TILE = E // NS
CCH = 2000
E_MAIN_TILE = E // NW
SUP = 2000
MCH = 80
NCHUNK = SUP // MCH
NBUF = 3
RECIP_CH = 1024

BN_H = 1000
BN_EPI = 2000


def _h_table(x, w_rel):
    def body(x_ref, w_ref, h_ref):
        for r in range(R):
            h_ref[:, r, :] = jnp.dot(x_ref[...], w_ref[r],
                                     preferred_element_type=jnp.float32)

    return pl.pallas_call(
        body,
        grid=(N // BN_H,),
        in_specs=[
            pl.BlockSpec((BN_H, D), lambda i: (i, 0)),
            pl.BlockSpec((R, D, D), lambda i: (0, 0, 0)),
        ],
        out_specs=pl.BlockSpec((BN_H, R, D), lambda i: (i, 0, 0)),
        out_shape=jax.ShapeDtypeStruct((N, R, D), jnp.float32),
    )(x, w_rel)


def _sc_counts(dst, typ):
    mesh = plsc.VectorSubcoreMesh(core_axis_name="c", subcore_axis_name="s",
                                  num_cores=NC, num_subcores=NS)

    def body(dst_hbm, typ_hbm, recip_hbm,
             cbuf_v, dstc0_v, dstc1_v, typc0_v, typc1_v, cidx0_v, cidx1_v,
             ones_v, cnt_sh, seml, semsc0, semsc1):
        c = lax.axis_index("c")
        s = lax.axis_index("s")

        zeros16 = jnp.zeros((LANES,), jnp.float32)

        def zero_cbuf(i, carry):
            cbuf_v[pl.ds(i * LANES, LANES)] = zeros16
            return carry
        lax.fori_loop(0, RECIP_CH // LANES, zero_cbuf, 0)

        def fill_ones(i, carry):
            ones_v[pl.ds(i * LANES, LANES)] = jnp.full((LANES,), 1.0,
                                                       jnp.float32)
            return carry
        lax.fori_loop(0, CCH // LANES, fill_ones, 0)

        for k in range(CNT_SLICE // RECIP_CH):
            pltpu.sync_copy(
                cbuf_v,
                cnt_sh.at[pl.ds(s * CNT_SLICE + k * RECIP_CH, RECIP_CH)])

        plsc.subcore_barrier()

        dstc = (dstc0_v, dstc1_v)
        typc = (typc0_v, typc1_v)
        cidx = (cidx0_v, cidx1_v)
        semsc = (semsc0, semsc1)
        nch = E_CNT_TILE // CCH
        base_t = s * E_CNT_TILE
        pltpu.async_copy(dst_hbm.at[pl.ds(base_t, CCH)], dstc0_v, seml)
        pltpu.async_copy(typ_hbm.at[pl.ds(base_t, CCH)], typc0_v, seml)
        scat = [None, None]
        for k in range(nch):
            p = k % 2
            pltpu.make_async_copy(dst_hbm.at[pl.ds(0, CCH)], dstc[p],
                                  seml).wait()
            pltpu.make_async_copy(typ_hbm.at[pl.ds(0, CCH)], typc[p],
                                  seml).wait()
            if k + 1 < nch:
                nb = base_t + (k + 1) * CCH
                pltpu.async_copy(dst_hbm.at[pl.ds(nb, CCH)],
                                 dstc[(k + 1) % 2], seml)
                pltpu.async_copy(typ_hbm.at[pl.ds(nb, CCH)],
                                 typc[(k + 1) % 2], seml)
            if scat[p] is not None:
                scat[p].wait()

            def cidx_iter(i, cc, p=p):
                dv = dstc[p][pl.ds(i * LANES, LANES)]
                tv = typc[p][pl.ds(i * LANES, LANES)]
                cidx[p][pl.ds(i * LANES, LANES)] = dv * R + tv
                return cc
            lax.fori_loop(0, CCH // LANES, cidx_iter, 0)
            scat[p] = pltpu.async_copy(ones_v, cnt_sh.at[cidx[p]],
                                       semsc[p], add=True)
        for d in scat:
            if d is not None:
                d.wait()

        plsc.subcore_barrier()

        for k in range(CNT_SLICE // RECIP_CH):
            off = s * CNT_SLICE + k * RECIP_CH
            pltpu.sync_copy(cnt_sh.at[pl.ds(off, RECIP_CH)], cbuf_v)

            def recip_iter(i, carry):
                v = cbuf_v[pl.ds(i * LANES, LANES)]
                cbuf_v[pl.ds(i * LANES, LANES)] = 1.0 / jnp.maximum(v, 1.0)
                return carry
            lax.fori_loop(0, RECIP_CH // LANES, recip_iter, 0)
            pltpu.sync_copy(cbuf_v, cnt_sh.at[pl.ds(off, RECIP_CH)])

        plsc.subcore_barrier()
        pltpu.sync_copy(
            cnt_sh.at[pl.ds(s * CNT_SLICE, CNT_SLICE)],
            recip_hbm.at[pl.ds(c * NR_PAD + s * CNT_SLICE, CNT_SLICE)])

    run = pl.kernel(
        body,
        out_type=jax.ShapeDtypeStruct((NC * NR_PAD,), jnp.float32),
        mesh=mesh,
        scratch_types=[
            pltpu.VMEM((RECIP_CH,), jnp.float32),
            pltpu.VMEM((CCH,), jnp.int32),
            pltpu.VMEM((CCH,), jnp.int32),
            pltpu.VMEM((CCH,), jnp.int32),
            pltpu.VMEM((CCH,), jnp.int32),
            pltpu.VMEM((CCH,), jnp.int32),
            pltpu.VMEM((CCH,), jnp.int32),
            pltpu.VMEM((CCH,), jnp.float32),
            pltpu.VMEM_SHARED((NR_PAD,), jnp.float32),
            pltpu.SemaphoreType.DMA,
            pltpu.SemaphoreType.DMA,
            pltpu.SemaphoreType.DMA,
        ],
    )
    return run(dst, typ)


def _sc_aggregate(src, dst, typ, h_table, recip):
    mesh = plsc.VectorSubcoreMesh(core_axis_name="c", subcore_axis_name="s",
                                  num_cores=NC, num_subcores=NS)

    def body(src_hbm, dst_hbm, typ_hbm, h_hbm, recip_hbm, mpart_hbm,
             srcc_v, dstc_v, typc_v, cidx_v, w_v,
             gidx_v, dsti_v, rows0_v, rows1_v, rows2_v,
             m_sh, semw, semg0, semg1, semg2, sems0, sems1, sems2, seme):
        c = lax.axis_index("c")
        s = lax.axis_index("s")
        g = c * NS + s
        rows = (rows0_v, rows1_v, rows2_v)
        semg = (semg0, semg1, semg2)
        sems = (sems0, sems1, sems2)

        zeros16 = jnp.zeros((LANES,), jnp.float32)

        def zero_rows(i, carry):
            for j in range(D // LANES):
                rows0_v[i, pl.ds(j * LANES, LANES)] = zeros16
            return carry
        lax.fori_loop(0, MCH, zero_rows, 0)

        m_row0 = s * M_SLICE
        for k in range(M_SLICE // MCH):
            pltpu.sync_copy(rows0_v,
                            m_sh.at[pl.ds(m_row0 + k * MCH, MCH), :])

        plsc.subcore_barrier()

        def scale_chunk(b, k):
            def scale_iter(i, cc):
                wv = w_v[pl.ds(k * MCH + i * LANES, LANES)]
                for j in range(LANES):
                    e = i * LANES + j
                    w = wv[j]
                    for q in range(D // LANES):
                        rows[b][e, pl.ds(q * LANES, LANES)] = (
                            rows[b][e, pl.ds(q * LANES, LANES)] * w)
                return cc
            lax.fori_loop(0, MCH // LANES, scale_iter, 0)

        n_sup = E_MAIN_TILE // SUP
        base0 = g * E_MAIN_TILE
        pltpu.async_copy(src_hbm.at[pl.ds(base0, SUP)], srcc_v, seme)
        pltpu.async_copy(dst_hbm.at[pl.ds(base0, SUP)], dstc_v, seme)
        pltpu.async_copy(typ_hbm.at[pl.ds(base0, SUP)], typc_v, seme)

        def superblock(sb, carry):
            pltpu.make_async_copy(src_hbm.at[pl.ds(0, SUP)], srcc_v,
                                  seme).wait()
            pltpu.make_async_copy(dst_hbm.at[pl.ds(0, SUP)], dstc_v,
                                  seme).wait()
            pltpu.make_async_copy(typ_hbm.at[pl.ds(0, SUP)], typc_v,
                                  seme).wait()

            coff = c * NR_PAD

            def idx_iter(k, cc):
                for j in range(MCH // LANES):
                    off = k * MCH + j * LANES
                    sv = srcc_v[pl.ds(off, LANES)]
                    dv = dstc_v[pl.ds(off, LANES)]
                    tv = typc_v[pl.ds(off, LANES)]
                    gidx_v[k, pl.ds(j * LANES, LANES)] = sv * R + tv
                    dsti_v[k, pl.ds(j * LANES, LANES)] = dv
                    cidx_v[pl.ds(off, LANES)] = dv * R + tv + coff
                return cc
            lax.fori_loop(0, NCHUNK, idx_iter, 0)

            @pl.when(sb + 1 < n_sup)
            def _prefetch():
                nbase = base0 + (sb + 1) * SUP
                pltpu.async_copy(src_hbm.at[pl.ds(nbase, SUP)], srcc_v, seme)
                pltpu.async_copy(dst_hbm.at[pl.ds(nbase, SUP)], dstc_v, seme)
                pltpu.async_copy(typ_hbm.at[pl.ds(nbase, SUP)], typc_v, seme)

            w_d = pltpu.async_copy(recip_hbm.at[cidx_v], w_v, semw)

            gather_d = [None] * NBUF
            scatter_d = [None] * NBUF
            gather_d[0] = pltpu.async_copy(
                h_hbm.at[gidx_v.at[0]], rows[0], semg[0])
            gather_d[1] = pltpu.async_copy(
                h_hbm.at[gidx_v.at[1]], rows[1], semg[1])
            w_d.wait()
            for k in range(NCHUNK):
                b = k % NBUF
                nb = (k + 2) % NBUF
                if k + 2 < NCHUNK:
                    if scatter_d[nb] is not None:
                        scatter_d[nb].wait()
                        scatter_d[nb] = None
                    gather_d[nb] = pltpu.async_copy(
                        h_hbm.at[gidx_v.at[k + 2]], rows[nb], semg[nb])
                gather_d[b].wait()
                scale_chunk(b, k)
                scatter_d[b] = pltpu.async_copy(
                    rows[b], m_sh.at[dsti_v.at[k]], sems[b], add=True)
            for d in scatter_d:
                if d is not None:
                    d.wait()
            return carry
        lax.fori_loop(0, n_sup, superblock, 0)

        plsc.subcore_barrier()

        pltpu.sync_copy(m_sh.at[pl.ds(m_row0, M_SLICE), :],
                        mpart_hbm.at[c, pl.ds(m_row0, M_SLICE), :])

    run = pl.kernel(
        body,
        out_type=jax.ShapeDtypeStruct((NC, N_PAD, D), jnp.float32),
        mesh=mesh,
        scratch_types=[
            pltpu.VMEM((SUP,), jnp.int32),
            pltpu.VMEM((SUP,), jnp.int32),
            pltpu.VMEM((SUP,), jnp.int32),
            pltpu.VMEM((SUP,), jnp.int32),
            pltpu.VMEM((SUP,), jnp.float32),
            pltpu.VMEM((NCHUNK, MCH), jnp.int32),
            pltpu.VMEM((NCHUNK, MCH), jnp.int32),
            pltpu.VMEM((MCH, D), jnp.float32),
            pltpu.VMEM((MCH, D), jnp.float32),
            pltpu.VMEM((MCH, D), jnp.float32),
            pltpu.VMEM_SHARED((N_PAD, D), jnp.float32),
            pltpu.SemaphoreType.DMA,
            pltpu.SemaphoreType.DMA,
            pltpu.SemaphoreType.DMA,
            pltpu.SemaphoreType.DMA,
            pltpu.SemaphoreType.DMA,
            pltpu.SemaphoreType.DMA,
            pltpu.SemaphoreType.DMA,
            pltpu.SemaphoreType.DMA,
        ],
    )
    return run(src, dst, typ, h_table, recip)


def _epilogue(x, w_root, bias, gamma, beta, m0, m1):
    def body(x_ref, wr_ref, b_ref, g_ref, be_ref, m0_ref, m1_ref, o_ref):
        x = x_ref[...]
        agg = (jnp.dot(x, wr_ref[...], preferred_element_type=jnp.float32)
               + b_ref[...] + m0_ref[...] + m1_ref[...])
        u = 0.5 * agg * (1.0 + lax.erf(agg * (2.0 ** -0.5)))
        y = u + x
        mean = jnp.mean(y, axis=-1, keepdims=True)
        yc = y - mean
        var = jnp.mean(yc * yc, axis=-1, keepdims=True)
        o_ref[...] = (yc * lax.rsqrt(var + 1e-5) * g_ref[...] + be_ref[...])

    return pl.pallas_call(
        body,
        grid=(N // BN_EPI,),
        in_specs=[
            pl.BlockSpec((BN_EPI, D), lambda i: (i, 0)),
            pl.BlockSpec((D, D), lambda i: (0, 0)),
            pl.BlockSpec((1, D), lambda i: (0, 0)),
            pl.BlockSpec((1, D), lambda i: (0, 0)),
            pl.BlockSpec((1, D), lambda i: (0, 0)),
            pl.BlockSpec((BN_EPI, D), lambda i: (i, 0)),
            pl.BlockSpec((BN_EPI, D), lambda i: (i, 0)),
        ],
        out_specs=pl.BlockSpec((BN_EPI, D), lambda i: (i, 0)),
        out_shape=jax.ShapeDtypeStruct((N, D), jnp.float32),
    )(x, w_root, bias.reshape(1, D), gamma.reshape(1, D),
      beta.reshape(1, D), m0, m1)


def kernel(node_states, edge_index, edge_type_ids, W_rel, W_root, bias,
           gamma, beta):
    src = edge_index[0]
    dst = edge_index[1]
    typ = edge_type_ids.astype(jnp.int32)

    h = _h_table(node_states, W_rel).reshape(N * R, D)
    recip = _sc_counts(dst, typ)
    mpart = _sc_aggregate(src, dst, typ, h, recip)
    return _epilogue(node_states, W_root, bias, gamma, beta,
                     mpart[0, :N], mpart[1, :N])


# speedup vs baseline: 44.8299x; 1.0232x over previous
import functools

import jax
import jax.numpy as jnp
from jax import lax
from jax.experimental import pallas as pl
from jax.experimental.pallas import tpu as pltpu
from jax.experimental.pallas import tpu_sc as plsc

NC = 2
NS = 16
NW = NC * NS
LANES = 16

N = 10000
E = 320000
D = 128
R = 8

N_PAD = 10112
NR_PAD = 81920
CNT_SLICE = NR_PAD // NS
M_SLICE = N_PAD // NS
E_CNT_TILE = E // NS
CCH = 2000
E_MAIN_TILE = E // NW
SUP = 2000
MCH = 80
NCHUNK = SUP // MCH
NBUF = 3
RECIP_CH = 1024

BN_H = 1000
BN_EPI = 2000


def _h_table(x, w_rel):
    def body(x_ref, w_ref, h_ref):
        for r in range(R):
            h_ref[:, r, :] = jnp.dot(x_ref[...], w_ref[r],
                                     preferred_element_type=jnp.float32)

    return pl.pallas_call(
        body,
        grid=(N // BN_H,),
        in_specs=[
            pl.BlockSpec((BN_H, D), lambda i: (i, 0)),
            pl.BlockSpec((R, D, D), lambda i: (0, 0, 0)),
        ],
        out_specs=pl.BlockSpec((BN_H, R, D), lambda i: (i, 0, 0)),
        out_shape=jax.ShapeDtypeStruct((N, R, D), jnp.float32),
    )(x, w_rel)


def _sc_counts(dst, typ):
    mesh = plsc.VectorSubcoreMesh(core_axis_name="c", subcore_axis_name="s",
                                  num_cores=NC, num_subcores=NS)

    def body(dst_hbm, typ_hbm, recip_hbm,
             cbuf_v, dstc0_v, dstc1_v, typc0_v, typc1_v, cidx0_v, cidx1_v,
             ones_v, cnt_sh, seml, semsc0, semsc1):
        c = lax.axis_index("c")
        s = lax.axis_index("s")

        zeros16 = jnp.zeros((LANES,), jnp.float32)

        def zero_cbuf(i, carry):
            cbuf_v[pl.ds(i * LANES, LANES)] = zeros16
            return carry
        lax.fori_loop(0, RECIP_CH // LANES, zero_cbuf, 0)

        def fill_ones(i, carry):
            ones_v[pl.ds(i * LANES, LANES)] = jnp.full((LANES,), 1.0,
                                                       jnp.float32)
            return carry
        lax.fori_loop(0, CCH // LANES, fill_ones, 0)

        for k in range(CNT_SLICE // RECIP_CH):
            pltpu.sync_copy(
                cbuf_v,
                cnt_sh.at[pl.ds(s * CNT_SLICE + k * RECIP_CH, RECIP_CH)])

        plsc.subcore_barrier()

        dstc = (dstc0_v, dstc1_v)
        typc = (typc0_v, typc1_v)
        cidx = (cidx0_v, cidx1_v)
        semsc = (semsc0, semsc1)
        nch = E_CNT_TILE // CCH
        base_t = s * E_CNT_TILE
        pltpu.async_copy(dst_hbm.at[pl.ds(base_t, CCH)], dstc0_v, seml)
        pltpu.async_copy(typ_hbm.at[pl.ds(base_t, CCH)], typc0_v, seml)
        scat = [None, None]
        for k in range(nch):
            p = k % 2
            pltpu.make_async_copy(dst_hbm.at[pl.ds(0, CCH)], dstc[p],
                                  seml).wait()
            pltpu.make_async_copy(typ_hbm.at[pl.ds(0, CCH)], typc[p],
                                  seml).wait()
            if k + 1 < nch:
                nb = base_t + (k + 1) * CCH
                pltpu.async_copy(dst_hbm.at[pl.ds(nb, CCH)],
                                 dstc[(k + 1) % 2], seml)
                pltpu.async_copy(typ_hbm.at[pl.ds(nb, CCH)],
                                 typc[(k + 1) % 2], seml)
            if scat[p] is not None:
                scat[p].wait()

            def cidx_iter(i, cc, p=p):
                dv = dstc[p][pl.ds(i * LANES, LANES)]
                tv = typc[p][pl.ds(i * LANES, LANES)]
                cidx[p][pl.ds(i * LANES, LANES)] = dv * R + tv
                return cc
            lax.fori_loop(0, CCH // LANES, cidx_iter, 0)
            scat[p] = pltpu.async_copy(ones_v, cnt_sh.at[cidx[p]],
                                       semsc[p], add=True)
        for d in scat:
            if d is not None:
                d.wait()

        plsc.subcore_barrier()

        for k in range(CNT_SLICE // RECIP_CH):
            off = s * CNT_SLICE + k * RECIP_CH
            pltpu.sync_copy(cnt_sh.at[pl.ds(off, RECIP_CH)], cbuf_v)

            def recip_iter(i, carry):
                v = cbuf_v[pl.ds(i * LANES, LANES)]
                cbuf_v[pl.ds(i * LANES, LANES)] = 1.0 / jnp.maximum(v, 1.0)
                return carry
            lax.fori_loop(0, RECIP_CH // LANES, recip_iter, 0)
            pltpu.sync_copy(cbuf_v, cnt_sh.at[pl.ds(off, RECIP_CH)])

        plsc.subcore_barrier()
        pltpu.sync_copy(
            cnt_sh.at[pl.ds(s * CNT_SLICE, CNT_SLICE)],
            recip_hbm.at[pl.ds(c * NR_PAD + s * CNT_SLICE, CNT_SLICE)])

    run = pl.kernel(
        body,
        out_type=jax.ShapeDtypeStruct((NC * NR_PAD,), jnp.float32),
        mesh=mesh,
        scratch_types=[
            pltpu.VMEM((RECIP_CH,), jnp.float32),
            pltpu.VMEM((CCH,), jnp.int32),
            pltpu.VMEM((CCH,), jnp.int32),
            pltpu.VMEM((CCH,), jnp.int32),
            pltpu.VMEM((CCH,), jnp.int32),
            pltpu.VMEM((CCH,), jnp.int32),
            pltpu.VMEM((CCH,), jnp.int32),
            pltpu.VMEM((CCH,), jnp.float32),
            pltpu.VMEM_SHARED((NR_PAD,), jnp.float32),
            pltpu.SemaphoreType.DMA,
            pltpu.SemaphoreType.DMA,
            pltpu.SemaphoreType.DMA,
        ],
    )
    return run(dst, typ)


def _sc_aggregate(src, dst, typ, h_table, recip):
    mesh = plsc.VectorSubcoreMesh(core_axis_name="c", subcore_axis_name="s",
                                  num_cores=NC, num_subcores=NS)

    def body(src_hbm, dst_hbm, typ_hbm, h_hbm, recip_hbm, mpart_hbm,
             srcc_v, dstc_v, typc_v, cidx_v, w0_v, w1_v, w2_v,
             gidx_v, dsti_v, rows0_v, rows1_v, rows2_v,
             m_sh, semw0, semw1, semw2, semg0, semg1, semg2,
             sems0, sems1, sems2, seme):
        c = lax.axis_index("c")
        s = lax.axis_index("s")
        g = c * NS + s
        rows = (rows0_v, rows1_v, rows2_v)
        wring = (w0_v, w1_v, w2_v)
        semg = (semg0, semg1, semg2)
        semw = (semw0, semw1, semw2)
        sems = (sems0, sems1, sems2)

        zeros16 = jnp.zeros((LANES,), jnp.float32)

        def zero_rows(i, carry):
            for j in range(D // LANES):
                rows0_v[i, pl.ds(j * LANES, LANES)] = zeros16
            return carry
        lax.fori_loop(0, MCH, zero_rows, 0)

        m_row0 = s * M_SLICE
        for k in range(M_SLICE // MCH):
            pltpu.sync_copy(rows0_v,
                            m_sh.at[pl.ds(m_row0 + k * MCH, MCH), :])
        if M_SLICE % MCH:
            pltpu.sync_copy(
                rows0_v.at[pl.ds(0, M_SLICE % MCH), :],
                m_sh.at[pl.ds(m_row0 + M_SLICE // MCH * MCH,
                              M_SLICE % MCH), :])

        plsc.subcore_barrier()

        def scale_chunk(b):
            def scale_iter(i, cc):
                wv = wring[b][pl.ds(i * LANES, LANES)]
                for j in range(LANES):
                    e = i * LANES + j
                    w = wv[j]
                    for q in range(D // LANES):
                        rows[b][e, pl.ds(q * LANES, LANES)] = (
                            rows[b][e, pl.ds(q * LANES, LANES)] * w)
                return cc
            lax.fori_loop(0, MCH // LANES, scale_iter, 0)

        n_sup = E_MAIN_TILE // SUP
        base0 = g * E_MAIN_TILE
        pltpu.async_copy(src_hbm.at[pl.ds(base0, SUP)], srcc_v, seme)
        pltpu.async_copy(dst_hbm.at[pl.ds(base0, SUP)], dstc_v, seme)
        pltpu.async_copy(typ_hbm.at[pl.ds(base0, SUP)], typc_v, seme)

        def superblock(sb, carry):
            pltpu.make_async_copy(src_hbm.at[pl.ds(0, SUP)], srcc_v,
                                  seme).wait()
            pltpu.make_async_copy(dst_hbm.at[pl.ds(0, SUP)], dstc_v,
                                  seme).wait()
            pltpu.make_async_copy(typ_hbm.at[pl.ds(0, SUP)], typc_v,
                                  seme).wait()

            coff = c * NR_PAD

            def idx_iter(k, cc):
                for j in range(MCH // LANES):
                    off = k * MCH + j * LANES
                    sv = srcc_v[pl.ds(off, LANES)]
                    dv = dstc_v[pl.ds(off, LANES)]
                    tv = typc_v[pl.ds(off, LANES)]
                    gidx_v[k, pl.ds(j * LANES, LANES)] = sv * R + tv
                    dsti_v[k, pl.ds(j * LANES, LANES)] = dv
                    cidx_v[k, pl.ds(j * LANES, LANES)] = dv * R + tv + coff
                return cc
            lax.fori_loop(0, NCHUNK, idx_iter, 0)

            @pl.when(sb + 1 < n_sup)
            def _prefetch():
                nbase = base0 + (sb + 1) * SUP
                pltpu.async_copy(src_hbm.at[pl.ds(nbase, SUP)], srcc_v, seme)
                pltpu.async_copy(dst_hbm.at[pl.ds(nbase, SUP)], dstc_v, seme)
                pltpu.async_copy(typ_hbm.at[pl.ds(nbase, SUP)], typc_v, seme)

            gather_d = [None] * NBUF
            wg_d = [None] * NBUF
            scatter_d = [None] * NBUF
            for k0 in range(2):
                gather_d[k0] = pltpu.async_copy(
                    h_hbm.at[gidx_v.at[k0]], rows[k0], semg[k0])
                wg_d[k0] = pltpu.async_copy(
                    recip_hbm.at[cidx_v.at[k0]], wring[k0], semw[k0])
            for k in range(NCHUNK):
                b = k % NBUF
                nb = (k + 2) % NBUF
                if k + 2 < NCHUNK:
                    if scatter_d[nb] is not None:
                        scatter_d[nb].wait()
                        scatter_d[nb] = None
                    gather_d[nb] = pltpu.async_copy(
                        h_hbm.at[gidx_v.at[k + 2]], rows[nb], semg[nb])
                    wg_d[nb] = pltpu.async_copy(
                        recip_hbm.at[cidx_v.at[k + 2]], wring[nb], semw[nb])
                gather_d[b].wait()
                wg_d[b].wait()
                scale_chunk(b)
                scatter_d[b] = pltpu.async_copy(
                    rows[b], m_sh.at[dsti_v.at[k]], sems[b], add=True)
            for d in scatter_d:
                if d is not None:
                    d.wait()
            return carry
        lax.fori_loop(0, n_sup, superblock, 0)

        plsc.subcore_barrier()

        pltpu.sync_copy(m_sh.at[pl.ds(m_row0, M_SLICE), :],
                        mpart_hbm.at[c, pl.ds(m_row0, M_SLICE), :])

    run = pl.kernel(
        body,
        out_type=jax.ShapeDtypeStruct((NC, N_PAD, D), jnp.float32),
        mesh=mesh,
        scratch_types=[
            pltpu.VMEM((SUP,), jnp.int32),
            pltpu.VMEM((SUP,), jnp.int32),
            pltpu.VMEM((SUP,), jnp.int32),
            pltpu.VMEM((NCHUNK, MCH), jnp.int32),
            pltpu.VMEM((MCH,), jnp.float32),
            pltpu.VMEM((MCH,), jnp.float32),
            pltpu.VMEM((MCH,), jnp.float32),
            pltpu.VMEM((NCHUNK, MCH), jnp.int32),
            pltpu.VMEM((NCHUNK, MCH), jnp.int32),
            pltpu.VMEM((MCH, D), jnp.float32),
            pltpu.VMEM((MCH, D), jnp.float32),
            pltpu.VMEM((MCH, D), jnp.float32),
            pltpu.VMEM_SHARED((N_PAD, D), jnp.float32),
            pltpu.SemaphoreType.DMA,
            pltpu.SemaphoreType.DMA,
            pltpu.SemaphoreType.DMA,
            pltpu.SemaphoreType.DMA,
            pltpu.SemaphoreType.DMA,
            pltpu.SemaphoreType.DMA,
            pltpu.SemaphoreType.DMA,
            pltpu.SemaphoreType.DMA,
            pltpu.SemaphoreType.DMA,
            pltpu.SemaphoreType.DMA,
        ],
    )
    return run(src, dst, typ, h_table, recip)


def _epilogue(x, w_root, bias, gamma, beta, m0, m1):
    def body(x_ref, wr_ref, b_ref, g_ref, be_ref, m0_ref, m1_ref, o_ref):
        x = x_ref[...]
        agg = (jnp.dot(x, wr_ref[...], preferred_element_type=jnp.float32)
               + b_ref[...] + m0_ref[...] + m1_ref[...])
        u = 0.5 * agg * (1.0 + lax.erf(agg * (2.0 ** -0.5)))
        y = u + x
        mean = jnp.mean(y, axis=-1, keepdims=True)
        yc = y - mean
        var = jnp.mean(yc * yc, axis=-1, keepdims=True)
        o_ref[...] = (yc * lax.rsqrt(var + 1e-5) * g_ref[...] + be_ref[...])

    return pl.pallas_call(
        body,
        grid=(N // BN_EPI,),
        in_specs=[
            pl.BlockSpec((BN_EPI, D), lambda i: (i, 0)),
            pl.BlockSpec((D, D), lambda i: (0, 0)),
            pl.BlockSpec((1, D), lambda i: (0, 0)),
            pl.BlockSpec((1, D), lambda i: (0, 0)),
            pl.BlockSpec((1, D), lambda i: (0, 0)),
            pl.BlockSpec((BN_EPI, D), lambda i: (i, 0)),
            pl.BlockSpec((BN_EPI, D), lambda i: (i, 0)),
        ],
        out_specs=pl.BlockSpec((BN_EPI, D), lambda i: (i, 0)),
        out_shape=jax.ShapeDtypeStruct((N, D), jnp.float32),
    )(x, w_root, bias.reshape(1, D), gamma.reshape(1, D),
      beta.reshape(1, D), m0, m1)


def kernel(node_states, edge_index, edge_type_ids, W_rel, W_root, bias,
           gamma, beta):
    src = edge_index[0]
    dst = edge_index[1]
    typ = edge_type_ids.astype(jnp.int32)

    h = _h_table(node_states, W_rel).reshape(N * R, D)
    recip = _sc_counts(dst, typ)
    mpart = _sc_aggregate(src, dst, typ, h, recip)
    return _epilogue(node_states, W_root, bias, gamma, beta,
                     mpart[0, :N], mpart[1, :N])
